# Initial kernel scaffold; baseline (speedup 1.0000x reference)
import jax
import jax.numpy as jnp
from jax.experimental import pallas as pl


def kernel(w1, g1, b1, a1, conv0_w, conv0_gamma, conv0_beta, conv0_alpha, conv1_w, conv1_gamma, conv1_beta, conv1_alpha, conv2_w, conv2_gamma, conv2_beta, conv2_alpha, conv3_w, conv3_gamma, conv3_beta, conv3_alpha, conv4_w, conv4_gamma, conv4_beta, conv4_alpha, fc1_w, fc1_gamma, fc1_beta, fc1_alpha, fc2_w, fc2_b, wc, x):
    raise NotImplementedError("write your pallas kernel here")


# speedup vs baseline: 2.2940x; 2.2940x over previous
import functools

import jax
import jax.numpy as jnp
from jax import lax
from jax.experimental import pallas as pl
from jax.experimental.pallas import tpu as pltpu

_EPS = 1e-5
_VMEM_LIMIT = 64 * 1024 * 1024


def _cp():
    return pltpu.CompilerParams(dimension_semantics=("parallel",),
                                vmem_limit_bytes=_VMEM_LIMIT)


def _pick(m, cap):
    d = min(m, cap)
    while m % d:
        d -= 1
    return d


def _conv_body(bsz, hi, wi, s, ho, wo, a_ref, sc_ref, sh_ref, al_ref, w_ref,
               y_ref, st_ref):
    cin = sc_ref.shape[1]
    cout = w_ref.shape[1]
    z = a_ref[...].astype(jnp.float32) * sc_ref[...].reshape(1, 1, cin) \
        + sh_ref[...].reshape(1, 1, cin)
    h = jnp.where(z >= 0.0, z, al_ref[0, 0] * z).astype(jnp.bfloat16)
    h = h.reshape(bsz, hi, wi, cin)
    if s == 1:
        taps = [h[:, kh:kh + ho, kw:kw + wo, :]
                for kh in range(3) for kw in range(3)]
    else:
        he, we = hi + (hi % 2), wi + (wi % 2)
        if we != wi:
            h = jnp.concatenate(
                [h, jnp.zeros((bsz, hi, 1, cin), h.dtype)], axis=2)
        if he != hi:
            h = jnp.concatenate(
                [h, jnp.zeros((bsz, 1, we, cin), h.dtype)], axis=1)
        hr = h.reshape(bsz, he // 2, 2, we, cin)
        ph = {}
        for r in (0, 1):
            hx = hr[:, :, r, :, :].reshape(bsz, he // 2, we // 2, 2, cin)
            for c in (0, 1):
                ph[(r, c)] = hx[:, :, :, c, :]
        taps = [ph[(kh % 2, kw % 2)][:, kh // 2:kh // 2 + ho,
                                     kw // 2:kw // 2 + wo, :]
                for kh in range(3) for kw in range(3)]
    col = jnp.concatenate(taps, axis=-1).reshape(bsz * ho * wo, 9 * cin)
    acc = jnp.dot(col, w_ref[...], preferred_element_type=jnp.float32)
    y_ref[...] = acc.astype(jnp.bfloat16).reshape(bsz, ho * wo, cout)
    st_ref[...] = jnp.concatenate(
        [jnp.sum(acc, axis=0, keepdims=True),
         jnp.sum(acc * acc, axis=0, keepdims=True)],
        axis=0).reshape(1, 2, cout)


def _conv3x3(act, sc, sh, al, w9, hi, wi, s, bcap):
    n, hw, cs = act.shape
    ho = (hi - 3) // s + 1
    wo = (wi - 3) // s + 1
    cin, cout = w9.shape[1], w9.shape[2]
    w = w9.reshape(9 * cin, cout)
    bsz = _pick(n, bcap)
    g = n // bsz
    y, st = pl.pallas_call(
        functools.partial(_conv_body, bsz, hi, wi, s, ho, wo),
        grid=(g,),
        in_specs=[pl.BlockSpec((bsz, hw, cs), lambda i: (i, 0, 0)),
                  pl.BlockSpec((1, cin), lambda i: (0, 0)),
                  pl.BlockSpec((1, cin), lambda i: (0, 0)),
                  pl.BlockSpec((1, 1), lambda i: (0, 0)),
                  pl.BlockSpec((9 * cin, cout), lambda i: (0, 0))],
        out_specs=(pl.BlockSpec((bsz, ho * wo, cout), lambda i: (i, 0, 0)),
                   pl.BlockSpec((1, 2, cout), lambda i: (i, 0, 0))),
        out_shape=(jax.ShapeDtypeStruct((n, ho * wo, cout), jnp.bfloat16),
                   jax.ShapeDtypeStruct((g, 2, cout), jnp.float32)),
        compiler_params=_cp(),
    )(act, sc, sh, al, w)
    return y, st.sum(axis=0)


def _mm_body(a_ref, sc_ref, sh_ref, al_ref, w_ref, z_ref, st_ref):
    z = a_ref[...].astype(jnp.float32) * sc_ref[...] + sh_ref[...]
    h = jnp.where(z >= 0.0, z, al_ref[0, 0] * z)
    acc = jnp.dot(h.astype(jnp.bfloat16), w_ref[...],
                  preferred_element_type=jnp.float32)
    z_ref[...] = acc.astype(z_ref.dtype)
    st_ref[...] = jnp.concatenate(
        [jnp.sum(acc, axis=0, keepdims=True),
         jnp.sum(acc * acc, axis=0, keepdims=True)],
        axis=0).reshape(1, 2, acc.shape[1])


def _norm_mm(a, sc, sh, al, w, tcap, out_dtype=jnp.bfloat16):
    m, c_in = a.shape
    c_out = w.shape[1]
    tm = _pick(m, tcap)
    g = m // tm
    z, st = pl.pallas_call(
        _mm_body,
        grid=(g,),
        in_specs=[pl.BlockSpec((tm, c_in), lambda i: (i, 0)),
                  pl.BlockSpec((1, c_in), lambda i: (0, 0)),
                  pl.BlockSpec((1, c_in), lambda i: (0, 0)),
                  pl.BlockSpec((1, 1), lambda i: (0, 0)),
                  pl.BlockSpec((c_in, c_out), lambda i: (0, 0))],
        out_specs=(pl.BlockSpec((tm, c_out), lambda i: (i, 0)),
                   pl.BlockSpec((1, 2, c_out), lambda i: (i, 0, 0))),
        out_shape=(jax.ShapeDtypeStruct((m, c_out), out_dtype),
                   jax.ShapeDtypeStruct((g, 2, c_out), jnp.float32)),
        compiler_params=_cp(),
    )(a, sc, sh, al, w)
    return z, st.sum(axis=0)


def _head_body(y_ref, sc_ref, sh_ref, al_ref, w_ref, b_ref, f_ref, o_ref):
    z = y_ref[...] * sc_ref[...] + sh_ref[...]
    h = jnp.where(z >= 0.0, z, al_ref[0, 0] * z)
    r = jnp.dot(h, w_ref[...], preferred_element_type=jnp.float32) + b_ref[...]
    f_ref[...] = r[:, 0:2]
    o_ref[...] = r[:, 2:12]


def _head(yf, sc, sh, al, wcat, bcat, tcap):
    n, c = yf.shape
    tm = _pick(n, tcap)
    g = n // tm
    return pl.pallas_call(
        _head_body,
        grid=(g,),
        in_specs=[pl.BlockSpec((tm, c), lambda i: (i, 0)),
                  pl.BlockSpec((1, c), lambda i: (0, 0)),
                  pl.BlockSpec((1, c), lambda i: (0, 0)),
                  pl.BlockSpec((1, 1), lambda i: (0, 0)),
                  pl.BlockSpec((c, 12), lambda i: (0, 0)),
                  pl.BlockSpec((1, 12), lambda i: (0, 0))],
        out_specs=(pl.BlockSpec((tm, 2), lambda i: (i, 0)),
                   pl.BlockSpec((tm, 10), lambda i: (i, 0))),
        out_shape=(jax.ShapeDtypeStruct((n, 2), jnp.float32),
                   jax.ShapeDtypeStruct((n, 10), jnp.float32)),
        compiler_params=_cp(),
    )(yf, sc, sh, al, wcat, bcat)


def _fold(st, count, gamma, beta):
    mean = st[0] / count
    var = st[1] / count - mean * mean
    s = gamma * lax.rsqrt(var + _EPS)
    return s.reshape(1, -1), (beta - mean * s).reshape(1, -1)


def kernel(w1, g1, b1, a1, conv0_w, conv0_gamma, conv0_beta, conv0_alpha,
           conv1_w, conv1_gamma, conv1_beta, conv1_alpha,
           conv2_w, conv2_gamma, conv2_beta, conv2_alpha,
           conv3_w, conv3_gamma, conv3_beta, conv3_alpha,
           conv4_w, conv4_gamma, conv4_beta, conv4_alpha,
           fc1_w, fc1_gamma, fc1_beta, fc1_alpha, fc2_w, fc2_b, wc, x):
    n = x.shape[0]
    xf = x.astype(jnp.float32).reshape(n, 28 * 28, 1)

    mx = jnp.mean(xf)
    vx = jnp.mean(xf * xf) - mx * mx
    inv = lax.rsqrt(w1 * w1 * vx + _EPS)
    sc = (w1 * g1 * inv).reshape(1, -1)
    sh = (b1 - w1 * mx * g1 * inv).reshape(1, -1)
    al = a1

    y0, st0 = _conv3x3(xf, sc, sh, al, conv0_w, 28, 28, 2, bcap=8)
    m0 = n * 169
    sc, sh = _fold(st0, m0, conv0_gamma, conv0_beta)

    z1, st1 = _norm_mm(y0.reshape(m0, 64), sc, sh, conv0_alpha, conv1_w,
                       tcap=2048)
    sc, sh = _fold(st1, m0, conv1_gamma, conv1_beta)

    y2, st2 = _conv3x3(z1.reshape(n, 169, 32), sc, sh, conv1_alpha, conv2_w,
                       13, 13, 2, bcap=64)
    m2 = n * 36
    sc, sh = _fold(st2, m2, conv2_gamma, conv2_beta)

    z3, st3 = _norm_mm(y2.reshape(m2, 64), sc, sh, conv2_alpha, conv3_w,
                       tcap=2048)
    sc, sh = _fold(st3, m2, conv3_gamma, conv3_beta)

    y4, st4 = _conv3x3(z3.reshape(n, 36, 32), sc, sh, conv3_alpha, conv4_w,
                       6, 6, 1, bcap=128)
    m4 = n * 16
    sc, sh = _fold(st4, m4, conv4_gamma, conv4_beta)

    yf, stf = _norm_mm(y4.reshape(n, 1024), jnp.tile(sc, (1, 16)),
                       jnp.tile(sh, (1, 16)), conv4_alpha, fc1_w,
                       tcap=1024, out_dtype=jnp.float32)
    scf, shf = _fold(stf, n, fc1_gamma, fc1_beta)

    w2 = fc2_w.astype(jnp.float32)
    wcat = jnp.concatenate([w2, w2 @ wc], axis=1)
    bcat = jnp.concatenate([fc2_b, fc2_b @ wc], axis=1)
    return _head(yf, scf, shf, fc1_alpha, wcat, bcat, tcap=2048)
